# baseline (speedup 1.0000x reference)
import jax
import jax.numpy as jnp
from jax.experimental import pallas as pl
from jax.experimental.pallas import tpu as pltpu

NS0 = 1024
PPS0 = 64
C = 256
M = 2048
SEG = 16
NBLK = NS0 // SEG
PTS = SEG * PPS0
BAND = SEG + 2
NPAR = SEG * 3

_INTERPRET = False
_HI = jax.lax.Precision.HIGHEST


def _assign_body(l2l_ref, pf_ref, hb_ref, rawa_ref, rawb_ref, spin_ref, *refs):
    g_refs = refs[:BAND]
    asg_ref, par_ref, o1_ref, o2_ref, spout_ref = refs[BAND:BAND + 5]
    sem1, sem2, sem3 = refs[BAND + 5:BAND + 8]
    b = pl.program_id(0)

    @pl.when(b == 0)
    def _start_copies():
        pltpu.make_async_copy(rawa_ref, o1_ref, sem1).start()
        pltpu.make_async_copy(rawb_ref, o2_ref, sem2).start()
        pltpu.make_async_copy(spin_ref, spout_ref, sem3).start()

    x = pf_ref[...].reshape(PTS, C)
    band = jnp.concatenate([g[0] for g in g_refs], axis=0)
    sims = jax.lax.dot_general(
        band, x, (((1,), (1,)), ((), ())), preferred_element_type=jnp.float32,
        precision=_HI,
    )

    jrow = jax.lax.broadcasted_iota(jnp.int32, (1, PTS), 1) // PPS0
    masks = [jrow == q for q in range(SEG)]
    s = []
    for k in range(3):
        acc = jnp.zeros((1, PTS), jnp.float32)
        for q in range(SEG):
            acc = acc + jnp.where(masks[q], sims[q + k:q + k + 1, :], 0.0)
        s.append(acc)
    b01 = s[0] >= s[1]
    v01 = jnp.where(b01, s[0], s[1])
    w2 = s[2] > v01
    w = jnp.where(w2, 2, jnp.where(b01, 0, 1))

    idv = [l2l_ref[jnp.clip(SEG * b - 1 + t, 0, NS0 - 1)] for t in range(BAND)]

    t3 = 3 * jrow + w
    rio = jax.lax.broadcasted_iota(jnp.int32, (NPAR, 1), 0)
    m48 = (rio == t3).astype(jnp.float32)

    cio = jax.lax.broadcasted_iota(jnp.int32, (1, NPAR), 1)
    tv = cio // 3 + cio % 3
    id48 = jnp.zeros((1, NPAR), jnp.float32)
    for t in range(BAND):
        id48 = id48 + jnp.where(tv == t, jnp.float32(1.0) * idv[t], 0.0)
    assigned = jax.lax.dot_general(
        id48, m48, (((1,), (0,)), ((), ())), preferred_element_type=jnp.float32,
        precision=_HI,
    )
    asg_ref[0] = assigned.astype(jnp.int32)

    hb = hb_ref[...].reshape(PTS, 3)
    rhs = jnp.concatenate([jnp.ones((PTS, 1), jnp.float32), hb], axis=1)
    out24 = jax.lax.dot_general(
        m48, rhs, (((1,), (0,)), ((), ())), preferred_element_type=jnp.float32,
        precision=_HI,
    )
    rio2 = jax.lax.broadcasted_iota(jnp.int32, (NPAR, 1), 0)
    tval = rio2 // 3 + rio2 % 3
    idcol = jnp.zeros((NPAR, 1), jnp.float32)
    for t in range(BAND):
        idcol = idcol + jnp.where(tval == t, jnp.float32(1.0) * idv[t], 0.0)
    par_ref[0] = jnp.concatenate([idcol, out24], axis=1)

    @pl.when(b == NBLK - 1)
    def _wait_copies():
        pltpu.make_async_copy(rawa_ref, o1_ref, sem1).wait()
        pltpu.make_async_copy(rawb_ref, o2_ref, sem2).wait()
        pltpu.make_async_copy(spin_ref, spout_ref, sem3).wait()


def _scatter_body(p_ref, o_ref):
    m = pl.program_id(0)
    P = p_ref[...]
    colf = (m * 128 + jax.lax.broadcasted_iota(jnp.int32, (1, 128), 1)).astype(jnp.float32)
    idc = jax.lax.slice(P, (0, 0), (NS0 * 3, 1))
    vals = jax.lax.slice(P, (0, 1), (NS0 * 3, 5))
    mask = (idc == colf).astype(jnp.float32)
    acc = jax.lax.dot_general(
        mask, vals, (((0,), (0,)), ((), ())), preferred_element_type=jnp.float32,
        precision=_HI,
    )
    coord = acc[:, 1:4] / jnp.maximum(acc[:, 0:1], 1.0)
    o_ref[...] = coord


def kernel(sp_center_feat, sp_center_coord, rawPoint_feat, hilbert_feat_coord,
           points_feat, points_coord, level0_to_level1_indices):
    del sp_center_coord, points_coord
    l2l = level0_to_level1_indices.astype(jnp.int32)
    hb3 = hilbert_feat_coord.reshape(NS0, PPS0, 3)
    sp3 = sp_center_feat.reshape(M, 1, C)
    rawa = rawPoint_feat.reshape(NS0, PPS0, C)
    rawb = rawPoint_feat.reshape(M, 32, C)

    def _g_spec(t):
        return pl.BlockSpec(
            (1, 1, C),
            lambda bb, l2l_ref, _t=t: (l2l_ref[jnp.clip(SEG * bb - 1 + _t, 0, NS0 - 1)], 0, 0),
        )

    grid_spec = pltpu.PrefetchScalarGridSpec(
        num_scalar_prefetch=1,
        grid=(NBLK,),
        in_specs=[
            pl.BlockSpec((SEG, PPS0, C), lambda bb, l2l_ref: (bb, 0, 0)),
            pl.BlockSpec((SEG, PPS0, 3), lambda bb, l2l_ref: (bb, 0, 0)),
            pl.BlockSpec(memory_space=pl.ANY),
            pl.BlockSpec(memory_space=pl.ANY),
            pl.BlockSpec(memory_space=pl.ANY),
        ] + [_g_spec(t) for t in range(BAND)],
        out_specs=[
            pl.BlockSpec((1, 1, PTS), lambda bb, l2l_ref: (bb, 0, 0)),
            pl.BlockSpec((1, NPAR, 5), lambda bb, l2l_ref: (bb, 0, 0)),
            pl.BlockSpec(memory_space=pl.ANY),
            pl.BlockSpec(memory_space=pl.ANY),
            pl.BlockSpec(memory_space=pl.ANY),
        ],
        scratch_shapes=[pltpu.SemaphoreType.DMA] * 3,
    )
    asg, par, points_feat_out, hilbert_feat_level1, sp_out = pl.pallas_call(
        _assign_body,
        grid_spec=grid_spec,
        out_shape=[
            jax.ShapeDtypeStruct((NBLK, 1, PTS), jnp.int32),
            jax.ShapeDtypeStruct((NBLK, NPAR, 5), jnp.float32),
            jax.ShapeDtypeStruct((NS0, PPS0, C), jnp.float32),
            jax.ShapeDtypeStruct((M, 32, C), jnp.float32),
            jax.ShapeDtypeStruct((M, C), jnp.float32),
        ],
        interpret=_INTERPRET,
    )(l2l, points_feat, hb3, rawa, rawb, sp_center_feat, *([sp3] * BAND))

    new_coord = pl.pallas_call(
        _scatter_body,
        grid=(M // 128,),
        in_specs=[pl.BlockSpec((NS0 * 3, 5), lambda m: (0, 0))],
        out_specs=pl.BlockSpec((128, 3), lambda m: (m, 0)),
        out_shape=jax.ShapeDtypeStruct((M, 3), jnp.float32),
        interpret=_INTERPRET,
    )(par.reshape(NS0 * 3, 5))

    point_assignments = asg.reshape(-1)
    return (point_assignments, sp_out, new_coord, points_feat_out,
            hilbert_feat_level1)


# speedup vs baseline: 20.9109x; 20.9109x over previous
import jax
import jax.numpy as jnp
from jax.experimental import pallas as pl
from jax.experimental.pallas import tpu as pltpu

NS0 = 1024
PPS0 = 64
C = 256
M = 2048
SEG = 16
NBLK = NS0 // SEG
PTS = SEG * PPS0
BAND = SEG + 2
NPAR = SEG * 3

_INTERPRET = False
_HI = jax.lax.Precision.HIGHEST


def _assign_body(l2l_ref, pf_ref, hb_ref, raw_ref, *refs):
    g_refs = refs[:BAND]
    asg_ref, par_ref, o1_ref, o2_ref = refs[BAND:BAND + 4]
    b = pl.program_id(0)
    xraw = raw_ref[...]
    o1_ref[...] = xraw.reshape(SEG, PPS0, C)
    o2_ref[...] = xraw.reshape(SEG * 2, 32, C)

    x = pf_ref[...].reshape(PTS, C)
    band = jnp.concatenate([g[0] for g in g_refs], axis=0)
    sims = jax.lax.dot_general(
        band, x, (((1,), (1,)), ((), ())), preferred_element_type=jnp.float32,
        precision=_HI,
    )

    jrow = jax.lax.broadcasted_iota(jnp.int32, (1, PTS), 1) // PPS0
    masks = [jrow == q for q in range(SEG)]
    s = []
    for k in range(3):
        acc = jnp.zeros((1, PTS), jnp.float32)
        for q in range(SEG):
            acc = acc + jnp.where(masks[q], sims[q + k:q + k + 1, :], 0.0)
        s.append(acc)
    b01 = s[0] >= s[1]
    v01 = jnp.where(b01, s[0], s[1])
    w2 = s[2] > v01
    w = jnp.where(w2, 2, jnp.where(b01, 0, 1))

    idv = [l2l_ref[jnp.clip(SEG * b - 1 + t, 0, NS0 - 1)] for t in range(BAND)]

    t3 = 3 * jrow + w
    rio = jax.lax.broadcasted_iota(jnp.int32, (NPAR, 1), 0)
    m48 = (rio == t3).astype(jnp.float32)

    cio = jax.lax.broadcasted_iota(jnp.int32, (1, NPAR), 1)
    tv = cio // 3 + cio % 3
    id48 = jnp.zeros((1, NPAR), jnp.float32)
    for t in range(BAND):
        id48 = id48 + jnp.where(tv == t, jnp.float32(1.0) * idv[t], 0.0)
    assigned = jax.lax.dot_general(
        id48, m48, (((1,), (0,)), ((), ())), preferred_element_type=jnp.float32,
        precision=_HI,
    )
    asg_ref[0] = assigned.astype(jnp.int32)

    hb = hb_ref[...].reshape(PTS, 3)
    rhs = jnp.concatenate([jnp.ones((PTS, 1), jnp.float32), hb], axis=1)
    out24 = jax.lax.dot_general(
        m48, rhs, (((1,), (0,)), ((), ())), preferred_element_type=jnp.float32,
        precision=_HI,
    )
    rio2 = jax.lax.broadcasted_iota(jnp.int32, (NPAR, 1), 0)
    tval = rio2 // 3 + rio2 % 3
    idcol = jnp.zeros((NPAR, 1), jnp.float32)
    for t in range(BAND):
        idcol = idcol + jnp.where(tval == t, jnp.float32(1.0) * idv[t], 0.0)
    par_ref[0] = jnp.concatenate([idcol, out24], axis=1)


def _scatter_body(p_ref, o_ref):
    m = pl.program_id(0)
    P = p_ref[...]
    colf = (m * 128 + jax.lax.broadcasted_iota(jnp.int32, (1, 128), 1)).astype(jnp.float32)
    idc = jax.lax.slice(P, (0, 0), (NS0 * 3, 1))
    vals = jax.lax.slice(P, (0, 1), (NS0 * 3, 5))
    mask = (idc == colf).astype(jnp.float32)
    acc = jax.lax.dot_general(
        mask, vals, (((0,), (0,)), ((), ())), preferred_element_type=jnp.float32,
        precision=_HI,
    )
    coord = acc[:, 1:4] / jnp.maximum(acc[:, 0:1], 1.0)
    o_ref[...] = coord


def kernel(sp_center_feat, sp_center_coord, rawPoint_feat, hilbert_feat_coord,
           points_feat, points_coord, level0_to_level1_indices):
    del sp_center_coord, points_coord
    l2l = level0_to_level1_indices.astype(jnp.int32)
    hb3 = hilbert_feat_coord.reshape(NS0, PPS0, 3)
    sp3 = sp_center_feat.reshape(M, 1, C)

    def _g_spec(t):
        return pl.BlockSpec(
            (1, 1, C),
            lambda bb, l2l_ref, _t=t: (l2l_ref[jnp.clip(SEG * bb - 1 + _t, 0, NS0 - 1)], 0, 0),
        )

    grid_spec = pltpu.PrefetchScalarGridSpec(
        num_scalar_prefetch=1,
        grid=(NBLK,),
        in_specs=[
            pl.BlockSpec((SEG, PPS0, C), lambda bb, l2l_ref: (bb, 0, 0)),
            pl.BlockSpec((SEG, PPS0, 3), lambda bb, l2l_ref: (bb, 0, 0)),
            pl.BlockSpec((PTS, C), lambda bb, l2l_ref: (bb, 0)),
        ] + [_g_spec(t) for t in range(BAND)],
        out_specs=[
            pl.BlockSpec((1, 1, PTS), lambda bb, l2l_ref: (bb, 0, 0)),
            pl.BlockSpec((1, NPAR, 5), lambda bb, l2l_ref: (bb, 0, 0)),
            pl.BlockSpec((SEG, PPS0, C), lambda bb, l2l_ref: (bb, 0, 0)),
            pl.BlockSpec((SEG * 2, 32, C), lambda bb, l2l_ref: (bb, 0, 0)),
        ],
    )
    asg, par, points_feat_out, hilbert_feat_level1 = pl.pallas_call(
        _assign_body,
        grid_spec=grid_spec,
        out_shape=[
            jax.ShapeDtypeStruct((NBLK, 1, PTS), jnp.int32),
            jax.ShapeDtypeStruct((NBLK, NPAR, 5), jnp.float32),
            jax.ShapeDtypeStruct((NS0, PPS0, C), jnp.float32),
            jax.ShapeDtypeStruct((M, 32, C), jnp.float32),
        ],
        interpret=_INTERPRET,
    )(l2l, points_feat, hb3, rawPoint_feat.reshape(NS0 * PPS0, C), *([sp3] * BAND))

    new_coord = pl.pallas_call(
        _scatter_body,
        grid=(M // 128,),
        in_specs=[pl.BlockSpec((NS0 * 3, 5), lambda m: (0, 0))],
        out_specs=pl.BlockSpec((128, 3), lambda m: (m, 0)),
        out_shape=jax.ShapeDtypeStruct((M, 3), jnp.float32),
        interpret=_INTERPRET,
    )(par.reshape(NS0 * 3, 5))

    point_assignments = asg.reshape(-1)
    return (point_assignments, sp_center_feat, new_coord, points_feat_out,
            hilbert_feat_level1)


# speedup vs baseline: 26.2637x; 1.2560x over previous
import jax
import jax.numpy as jnp
from jax.experimental import pallas as pl
from jax.experimental.pallas import tpu as pltpu

NS0 = 1024
PPS0 = 64
C = 256
M = 2048
SEG = 32
NBLK = NS0 // SEG
PTS = SEG * PPS0
BAND = SEG + 2
NPAR = SEG * 3

_INTERPRET = False
_HI = jax.lax.Precision.HIGHEST


def _split_dot(onehot, vals, dims):
    hi = vals.astype(jnp.bfloat16).astype(jnp.float32)
    lo = vals - hi
    a = jax.lax.dot_general(onehot, hi, dims, preferred_element_type=jnp.float32)
    b = jax.lax.dot_general(onehot, lo, dims, preferred_element_type=jnp.float32)
    return a + b


def _assign_body(l2l_ref, pf_ref, hb_ref, raw_ref, *refs):
    g_refs = refs[:BAND]
    asg_ref, par_ref, o1_ref, o2_ref = refs[BAND:BAND + 4]
    b = pl.program_id(0)
    xraw = raw_ref[...]
    o1_ref[...] = xraw.reshape(SEG, PPS0, C)
    o2_ref[...] = xraw.reshape(SEG * 2, 32, C)

    x = pf_ref[...].reshape(PTS, C)
    band = jnp.concatenate([g[0] for g in g_refs], axis=0)
    dimsnt = (((1,), (1,)), ((), ()))
    b_hi = band.astype(jnp.bfloat16).astype(jnp.float32)
    b_lo = band - b_hi
    x_hi = x.astype(jnp.bfloat16).astype(jnp.float32)
    x_lo = x - x_hi
    sims = (
        jax.lax.dot_general(b_hi, x_hi, dimsnt, preferred_element_type=jnp.float32)
        + jax.lax.dot_general(b_hi, x_lo, dimsnt, preferred_element_type=jnp.float32)
        + jax.lax.dot_general(b_lo, x_hi, dimsnt, preferred_element_type=jnp.float32)
    )

    jrow = jax.lax.broadcasted_iota(jnp.int32, (1, PTS), 1) // PPS0
    masks = [jrow == q for q in range(SEG)]
    s = []
    for k in range(3):
        acc = jnp.zeros((1, PTS), jnp.float32)
        for q in range(SEG):
            acc = acc + jnp.where(masks[q], sims[q + k:q + k + 1, :], 0.0)
        s.append(acc)
    b01 = s[0] >= s[1]
    v01 = jnp.where(b01, s[0], s[1])
    w2 = s[2] > v01
    w = jnp.where(w2, 2, jnp.where(b01, 0, 1))

    idv = [l2l_ref[jnp.clip(SEG * b - 1 + t, 0, NS0 - 1)] for t in range(BAND)]

    t3 = 3 * jrow + w
    rio = jax.lax.broadcasted_iota(jnp.int32, (NPAR, 1), 0)
    m48 = (rio == t3).astype(jnp.float32)

    cio = jax.lax.broadcasted_iota(jnp.int32, (1, NPAR), 1)
    tv = cio // 3 + cio % 3
    id48 = jnp.zeros((1, NPAR), jnp.float32)
    for t in range(BAND):
        id48 = id48 + jnp.where(tv == t, jnp.float32(1.0) * idv[t], 0.0)
    id_hi = id48.astype(jnp.bfloat16).astype(jnp.float32)
    id_lo = id48 - id_hi
    dims1 = (((1,), (0,)), ((), ()))
    assigned = (
        jax.lax.dot_general(id_hi, m48, dims1, preferred_element_type=jnp.float32)
        + jax.lax.dot_general(id_lo, m48, dims1, preferred_element_type=jnp.float32)
    )
    asg_ref[0] = assigned.astype(jnp.int32)

    hb = hb_ref[...].reshape(PTS, 3)
    rhs = jnp.concatenate([jnp.ones((PTS, 1), jnp.float32), hb], axis=1)
    out24 = _split_dot(m48, rhs, (((1,), (0,)), ((), ())))
    rio2 = jax.lax.broadcasted_iota(jnp.int32, (NPAR, 1), 0)
    tval = rio2 // 3 + rio2 % 3
    idcol = jnp.zeros((NPAR, 1), jnp.float32)
    for t in range(BAND):
        idcol = idcol + jnp.where(tval == t, jnp.float32(1.0) * idv[t], 0.0)
    par_ref[0] = jnp.concatenate([idcol, out24], axis=1)


def _scatter_body(p_ref, o_ref):
    m = pl.program_id(0)
    P = p_ref[...]
    colf = (m * 128 + jax.lax.broadcasted_iota(jnp.int32, (1, 128), 1)).astype(jnp.float32)
    idc = jax.lax.slice(P, (0, 0), (NS0 * 3, 1))
    vals = jax.lax.slice(P, (0, 1), (NS0 * 3, 5))
    mask = (idc == colf).astype(jnp.float32)
    acc = _split_dot(mask, vals, (((0,), (0,)), ((), ())))
    coord = acc[:, 1:4] / jnp.maximum(acc[:, 0:1], 1.0)
    o_ref[...] = coord


def kernel(sp_center_feat, sp_center_coord, rawPoint_feat, hilbert_feat_coord,
           points_feat, points_coord, level0_to_level1_indices):
    del sp_center_coord, points_coord
    l2l = level0_to_level1_indices.astype(jnp.int32)
    hb3 = hilbert_feat_coord.reshape(NS0, PPS0, 3)
    sp3 = sp_center_feat.reshape(M, 1, C)

    def _g_spec(t):
        return pl.BlockSpec(
            (1, 1, C),
            lambda bb, l2l_ref, _t=t: (l2l_ref[jnp.clip(SEG * bb - 1 + _t, 0, NS0 - 1)], 0, 0),
        )

    grid_spec = pltpu.PrefetchScalarGridSpec(
        num_scalar_prefetch=1,
        grid=(NBLK,),
        in_specs=[
            pl.BlockSpec((SEG, PPS0, C), lambda bb, l2l_ref: (bb, 0, 0)),
            pl.BlockSpec((SEG, PPS0, 3), lambda bb, l2l_ref: (bb, 0, 0)),
            pl.BlockSpec((PTS, C), lambda bb, l2l_ref: (bb, 0)),
        ] + [_g_spec(t) for t in range(BAND)],
        out_specs=[
            pl.BlockSpec((1, 1, PTS), lambda bb, l2l_ref: (bb, 0, 0)),
            pl.BlockSpec((1, NPAR, 5), lambda bb, l2l_ref: (bb, 0, 0)),
            pl.BlockSpec((SEG, PPS0, C), lambda bb, l2l_ref: (bb, 0, 0)),
            pl.BlockSpec((SEG * 2, 32, C), lambda bb, l2l_ref: (bb, 0, 0)),
        ],
    )
    asg, par, points_feat_out, hilbert_feat_level1 = pl.pallas_call(
        _assign_body,
        grid_spec=grid_spec,
        out_shape=[
            jax.ShapeDtypeStruct((NBLK, 1, PTS), jnp.int32),
            jax.ShapeDtypeStruct((NBLK, NPAR, 5), jnp.float32),
            jax.ShapeDtypeStruct((NS0, PPS0, C), jnp.float32),
            jax.ShapeDtypeStruct((M, 32, C), jnp.float32),
        ],
        interpret=_INTERPRET,
    )(l2l, points_feat, hb3, rawPoint_feat.reshape(NS0 * PPS0, C), *([sp3] * BAND))

    new_coord = pl.pallas_call(
        _scatter_body,
        grid=(M // 128,),
        in_specs=[pl.BlockSpec((NS0 * 3, 5), lambda m: (0, 0))],
        out_specs=pl.BlockSpec((128, 3), lambda m: (m, 0)),
        out_shape=jax.ShapeDtypeStruct((M, 3), jnp.float32),
        interpret=_INTERPRET,
    )(par.reshape(NS0 * 3, 5))

    point_assignments = asg.reshape(-1)
    return (point_assignments, sp_center_feat, new_coord, points_feat_out,
            hilbert_feat_level1)
